# baseline (speedup 1.0000x reference)
import functools

import jax
import jax.numpy as jnp
from jax import lax
from jax.experimental import pallas as pl
from jax.experimental.pallas import tpu as pltpu
from jax.experimental.pallas import tpu_sc as plsc

_NC, _NS = 2, 16
_NW = _NC * _NS
_ROWS = 8 * 2048
_C = 512
_RPW = _ROWS // _NW
_R = 128
_NCHUNK = _RPW // _R


def _make_sc_repeat():
    mesh = plsc.VectorSubcoreMesh(core_axis_name="c", subcore_axis_name="s")

    @functools.partial(
        pl.kernel,
        mesh=mesh,
        out_type=jax.ShapeDtypeStruct((_ROWS, 2, _C), jnp.float32),
        scratch_types=[
            pltpu.VMEM((_R, 1, _C), jnp.float32),
        ],
    )
    def sc_repeat(x_hbm, o_hbm, buf):
        w = lax.axis_index("s") * _NC + lax.axis_index("c")
        base = w * _RPW

        @pl.loop(0, _NCHUNK)
        def _(i):
            r0 = base + i * _R
            pltpu.sync_copy(x_hbm.at[pl.ds(r0, _R)], buf)
            pltpu.sync_copy(buf, o_hbm.at[pl.ds(r0, _R), pl.ds(0, 1)])
            pltpu.sync_copy(buf, o_hbm.at[pl.ds(r0, _R), pl.ds(1, 1)])

    return sc_repeat


_sc_repeat = _make_sc_repeat()


def kernel(x, duration):
    del duration
    B, T, C = x.shape
    x3 = x.reshape(B * T, 1, C)
    out = _sc_repeat(x3)
    return out.reshape(B, 2 * T, C)
